# baseline (speedup 1.0000x reference)
import functools

import jax
import jax.numpy as jnp
from jax import lax
from jax.experimental import pallas as pl
from jax.experimental.pallas import tpu as pltpu
from jax.experimental.pallas import tpu_sc as plsc

N = 100000
E = 300000
D = 128

NC, NS = 2, 16
EP = 311296
ER = EP // 128
RPT = ER // NS
PAD_COL = 100223
C = 12544
NPASS = 4
ACC_ROWS = 12800
S_P = 2 * NPASS * C
DEG_P = 100352
DPT = DEG_P // NS
CH = 8
NCHUNK = RPT // CH
LIST_CAP = 1280
GB = 64
P_C0 = 8

_mesh = plsc.VectorSubcoreMesh(
    core_axis_name="c", subcore_axis_name="s", num_cores=NC, num_subcores=NS)


@functools.partial(
    pl.kernel,
    out_type=[jax.ShapeDtypeStruct((DEG_P,), jnp.float32),
              jax.ShapeDtypeStruct((DEG_P,), jnp.float32)],
    mesh=_mesh,
    compiler_params=pltpu.CompilerParams(needs_layout_passes=False),
    scratch_types=[
        pltpu.VMEM((RPT, 128), jnp.int32),
        pltpu.VMEM((128,), jnp.float32),
        pltpu.VMEM((DPT,), jnp.float32),
        pltpu.VMEM_SHARED((DEG_P,), jnp.float32),
    ],
)
def _deg_kernel(col0_hbm, col1_hbm, deg0_hbm, deg1_hbm,
                colbuf, ones_v, zero_v, acc):
    c = lax.axis_index("c")
    s = lax.axis_index("s")

    def fill_ones(i, _):
        ones_v[pl.ds(i * 16, 16)] = jnp.full((16,), 1.0, jnp.float32)
        return 0
    lax.fori_loop(0, 8, fill_ones, 0)

    def fill_zero(i, _):
        zero_v[pl.ds(i * 16, 16)] = jnp.zeros((16,), jnp.float32)
        return 0
    lax.fori_loop(0, DPT // 16, fill_zero, 0)
    pltpu.sync_copy(zero_v, acc.at[pl.ds(s * DPT, DPT)])

    @pl.when(c == 0)
    def _():
        pltpu.sync_copy(col0_hbm.at[pl.ds(s * RPT, RPT)], colbuf)

    @pl.when(c == 1)
    def _():
        pltpu.sync_copy(col1_hbm.at[pl.ds(s * RPT, RPT)], colbuf)

    plsc.subcore_barrier()

    def body(j, _):
        pltpu.sync_copy(ones_v, acc.at[colbuf.at[j]], add=True)
        return 0
    lax.fori_loop(0, RPT, body, 0)

    plsc.subcore_barrier()

    @pl.when(c == 0)
    def _():
        pltpu.sync_copy(acc.at[pl.ds(s * DPT, DPT)],
                        deg0_hbm.at[pl.ds(s * DPT, DPT)])

    @pl.when(c == 1)
    def _():
        pltpu.sync_copy(acc.at[pl.ds(s * DPT, DPT)],
                        deg1_hbm.at[pl.ds(s * DPT, DPT)])


@functools.partial(
    pl.kernel,
    out_type=[jax.ShapeDtypeStruct((S_P, D), jnp.float32),
              jax.ShapeDtypeStruct((S_P, D), jnp.float32)],
    mesh=_mesh,
    compiler_params=pltpu.CompilerParams(needs_layout_passes=False),
    scratch_types=[
        pltpu.VMEM((2, CH, 128), jnp.int32),
        pltpu.VMEM((2, CH, 128), jnp.int32),
        pltpu.VMEM((LIST_CAP,), jnp.int32),
        pltpu.VMEM((LIST_CAP,), jnp.int32),
        pltpu.VMEM((2, GB), jnp.int32),
        pltpu.VMEM((2, GB, D), jnp.float32),
        pltpu.VMEM_SHARED((ACC_ROWS, D), jnp.float32),
        pltpu.SemaphoreType.DMA,
        pltpu.SemaphoreType.DMA,
        pltpu.SemaphoreType.DMA,
        pltpu.SemaphoreType.DMA,
        pltpu.SemaphoreType.DMA,
        pltpu.SemaphoreType.DMA,
        pltpu.SemaphoreType.DMA,
    ],
)
def _scatter_kernel(y0_hbm, y1_hbm, col0_hbm, row0_hbm, col1_hbm, row1_hbm,
                    s0_hbm, s1_hbm,
                    colbuf, rowbuf, collist, rowlist, colstage, gbuf, acc,
                    cb_sem, rb_sem, g_sem0, g_sem1, s_sem0, s_sem1, z_sem):
    c = lax.axis_index("c")
    s = lax.axis_index("s")
    iota = lax.iota(jnp.int32, 16)
    mall = jnp.ones((16,), jnp.bool_)
    spt = s * (ACC_ROWS // NS)
    g_sems = (g_sem0, g_sem1)
    s_sems = (s_sem0, s_sem1)

    for rel in range(2):
        col_hbm = (col0_hbm, col1_hbm)[rel]
        row_hbm = (row0_hbm, row1_hbm)[rel]
        y_hbm = (y0_hbm, y1_hbm)[rel]
        out_hbm = (s0_hbm, s1_hbm)[rel]

        trip = jnp.where(c == 0, P_C0, 2 * NPASS - P_C0)
        start = jnp.where(c == 0, 0, P_C0)

        def one_pass(p, _):
            base = (start + p) * C

            with jax.named_scope("sc_zero"):
                def zg(i, _):
                    for sl in range(2):
                        for k in range(8):
                            gbuf[sl, i, pl.ds(k * 16, 16)] = \
                                jnp.zeros((16,), jnp.float32)
                    return 0
                lax.fori_loop(0, GB, zg, 0)
                zdescs = [pltpu.make_async_copy(
                    gbuf.at[0], acc.at[pl.ds(spt + k * GB, GB)], z_sem)
                    for k in range(12)]
                zdescs.append(pltpu.make_async_copy(
                    gbuf.at[0].at[pl.ds(0, 32)],
                    acc.at[pl.ds(spt + 12 * GB, 32)], z_sem))
                for d in zdescs:
                    d.start()
                for d in zdescs:
                    d.wait()
            for sl in range(2):
                for k in range(4):
                    colstage[sl, pl.ds(k * 16, 16)] = \
                        jnp.full((16,), C, jnp.int32)
                pltpu.make_async_copy(
                    gbuf.at[sl], acc.at[colstage.at[sl]],
                    s_sems[sl]).start(add=True)
            plsc.subcore_barrier()

            def stage_and_gather(off, sl):
                pltpu.make_async_copy(
                    gbuf.at[sl], acc.at[colstage.at[sl]],
                    s_sems[sl]).wait()
                for k in range(4):
                    colstage[sl, pl.ds(k * 16, 16)] = \
                        plsc.load_gather(collist, [off + k * 16 + iota])
                pltpu.async_copy(
                    y_hbm.at[rowlist.at[pl.ds(off, GB)]],
                    gbuf.at[sl], g_sems[sl])

            def scatter(sl):
                pltpu.make_async_copy(
                    y_hbm.at[rowlist.at[pl.ds(0, GB)]],
                    gbuf.at[sl], g_sems[sl]).wait()
                pltpu.make_async_copy(
                    gbuf.at[sl], acc.at[colstage.at[sl]],
                    s_sems[sl]).start(add=True)

            def drain_pair(b, _):
                off = b * (2 * GB)
                stage_and_gather(off, 0)
                stage_and_gather(off + GB, 1)
                scatter(0)
                scatter(1)
                return 0

            def chunk_body(ch, n):
                par = lax.rem(ch, 2)
                nxt = s * RPT + (ch + 1) * CH

                @pl.when(ch + 1 < NCHUNK)
                def _():
                    pltpu.async_copy(col_hbm.at[pl.ds(nxt, CH)],
                                     colbuf.at[1 - par], cb_sem)
                    pltpu.async_copy(row_hbm.at[pl.ds(nxt, CH)],
                                     rowbuf.at[1 - par], rb_sem)

                def crow(j, n):
                    for i in range(8):
                        cv = colbuf[par, j, pl.ds(i * 16, 16)]
                        rv = rowbuf[par, j, pl.ds(i * 16, 16)]
                        m = (cv >= base) & (cv < base + C)
                        mi = m.astype(jnp.int32)
                        pos = plsc.cumsum(mi)
                        idx = n + pos - 1
                        plsc.store_scatter(collist, [idx], cv - base, mask=m)
                        plsc.store_scatter(rowlist, [idx], rv, mask=m)
                        n = n + pos[15]
                    return n
                with jax.named_scope("sc_compact"):
                    n = lax.fori_loop(0, CH, crow, n)
                nb = n // (2 * GB)
                with jax.named_scope("sc_drain"):
                    lax.fori_loop(0, nb, drain_pair, 0)
                roff = nb * (2 * GB)
                for k in range(8):
                    v = plsc.load_gather(collist, [roff + k * 16 + iota])
                    w = plsc.load_gather(rowlist, [roff + k * 16 + iota])
                    plsc.store_scatter(collist, [k * 16 + iota], v, mask=mall)
                    plsc.store_scatter(rowlist, [k * 16 + iota], w, mask=mall)

                @pl.when(ch + 1 < NCHUNK)
                def _():
                    pltpu.make_async_copy(col_hbm.at[pl.ds(nxt, CH)],
                                          colbuf.at[1 - par], cb_sem).wait()
                    pltpu.make_async_copy(row_hbm.at[pl.ds(nxt, CH)],
                                          rowbuf.at[1 - par], rb_sem).wait()
                return n - roff

            pltpu.sync_copy(col_hbm.at[pl.ds(s * RPT, CH)], colbuf.at[0])
            pltpu.sync_copy(row_hbm.at[pl.ds(s * RPT, CH)], rowbuf.at[0])
            n = lax.fori_loop(0, NCHUNK, chunk_body, 0)

            for k in range(8):
                plsc.store_scatter(collist, [n + k * 16 + iota],
                                   jnp.full((16,), C, jnp.int32), mask=mall)
                plsc.store_scatter(rowlist, [n + k * 16 + iota],
                                   jnp.zeros((16,), jnp.int32), mask=mall)
            drain_pair(0, 0)
            for sl in range(2):
                pltpu.make_async_copy(
                    gbuf.at[sl], acc.at[colstage.at[sl]],
                    s_sems[sl]).wait()
            plsc.subcore_barrier()
            with jax.named_scope("sc_writeout"):
                woff = s * (C // NS)
                pltpu.sync_copy(acc.at[pl.ds(woff, C // NS)],
                                out_hbm.at[pl.ds(base + woff, C // NS)])
            plsc.subcore_barrier()
            return 0
        lax.fori_loop(0, trip, one_pass, 0)


BR = 1024
GRID = (DEG_P // BR,)


def _mm_l1(deg0, deg1, emb, w0, w1):
    def body(deg0_ref, deg1_ref, emb_ref, w0_ref, w1_ref, y0_ref, y1_ref):
        dinv0 = lax.rsqrt(deg0_ref[...] + 1.0)
        dinv1 = lax.rsqrt(deg1_ref[...] + 1.0)
        x = emb_ref[...]
        y0_ref[...] = jnp.dot(x, w0_ref[...],
                              preferred_element_type=jnp.float32) * dinv0
        y1_ref[...] = jnp.dot(x, w1_ref[...],
                              preferred_element_type=jnp.float32) * dinv1
    return pl.pallas_call(
        body,
        grid=GRID,
        in_specs=[
            pl.BlockSpec((BR, 1), lambda i: (i, 0)),
            pl.BlockSpec((BR, 1), lambda i: (i, 0)),
            pl.BlockSpec((BR, D), lambda i: (i, 0)),
            pl.BlockSpec((D, D), lambda i: (0, 0)),
            pl.BlockSpec((D, D), lambda i: (0, 0)),
        ],
        out_specs=[pl.BlockSpec((BR, D), lambda i: (i, 0))] * 2,
        out_shape=[jax.ShapeDtypeStruct((N, D), jnp.float32)] * 2,
    )(deg0, deg1, emb, w0, w1)


def _mm_l2(deg0, deg1, s0, y0, s1, y1, b0, b1, w0, w1):
    def body(deg0_ref, deg1_ref, s0_ref, y0_ref, s1_ref, y1_ref,
             b0_ref, b1_ref, w0_ref, w1_ref, o0_ref, o1_ref):
        dinv0 = lax.rsqrt(deg0_ref[...] + 1.0)
        dinv1 = lax.rsqrt(deg1_ref[...] + 1.0)
        h = 0.5 * (dinv0 * (s0_ref[...] + y0_ref[...]) + b0_ref[...]
                   + dinv1 * (s1_ref[...] + y1_ref[...]) + b1_ref[...])
        h = jnp.maximum(h, 0.0)
        o0_ref[...] = jnp.dot(h, w0_ref[...],
                              preferred_element_type=jnp.float32) * dinv0
        o1_ref[...] = jnp.dot(h, w1_ref[...],
                              preferred_element_type=jnp.float32) * dinv1
    return pl.pallas_call(
        body,
        grid=GRID,
        in_specs=[
            pl.BlockSpec((BR, 1), lambda i: (i, 0)),
            pl.BlockSpec((BR, 1), lambda i: (i, 0)),
            pl.BlockSpec((BR, D), lambda i: (i, 0)),
            pl.BlockSpec((BR, D), lambda i: (i, 0)),
            pl.BlockSpec((BR, D), lambda i: (i, 0)),
            pl.BlockSpec((BR, D), lambda i: (i, 0)),
            pl.BlockSpec((1, D), lambda i: (0, 0)),
            pl.BlockSpec((1, D), lambda i: (0, 0)),
            pl.BlockSpec((D, D), lambda i: (0, 0)),
            pl.BlockSpec((D, D), lambda i: (0, 0)),
        ],
        out_specs=[pl.BlockSpec((BR, D), lambda i: (i, 0))] * 2,
        out_shape=[jax.ShapeDtypeStruct((N, D), jnp.float32)] * 2,
    )(deg0, deg1, s0, y0, s1, y1, b0, b1, w0, w1)


def _final(deg0, deg1, s0, y0, s1, y1, b0, b1):
    def body(deg0_ref, deg1_ref, s0_ref, y0_ref, s1_ref, y1_ref,
             b0_ref, b1_ref, o_ref):
        dinv0 = lax.rsqrt(deg0_ref[...] + 1.0)
        dinv1 = lax.rsqrt(deg1_ref[...] + 1.0)
        o_ref[...] = 0.5 * (dinv0 * (s0_ref[...] + y0_ref[...]) + b0_ref[...]
                            + dinv1 * (s1_ref[...] + y1_ref[...])
                            + b1_ref[...])
    return pl.pallas_call(
        body,
        grid=GRID,
        in_specs=[
            pl.BlockSpec((BR, 1), lambda i: (i, 0)),
            pl.BlockSpec((BR, 1), lambda i: (i, 0)),
            pl.BlockSpec((BR, D), lambda i: (i, 0)),
            pl.BlockSpec((BR, D), lambda i: (i, 0)),
            pl.BlockSpec((BR, D), lambda i: (i, 0)),
            pl.BlockSpec((BR, D), lambda i: (i, 0)),
            pl.BlockSpec((1, D), lambda i: (0, 0)),
            pl.BlockSpec((1, D), lambda i: (0, 0)),
        ],
        out_specs=pl.BlockSpec((BR, D), lambda i: (i, 0)),
        out_shape=jax.ShapeDtypeStruct((N, D), jnp.float32),
    )(deg0, deg1, s0, y0, s1, y1, b0, b1)


def kernel(edge_index_r0, edge_index_r1, emb, W1_r0, b1_r0, W1_r1, b1_r1,
           W2_r0, b2_r0, W2_r1, b2_r1):
    pad_col = jnp.full((EP - E,), PAD_COL, jnp.int32)
    pad_row = jnp.zeros((EP - E,), jnp.int32)
    col0 = jnp.concatenate([edge_index_r0[1], pad_col]).reshape(ER, 128)
    row0 = jnp.concatenate([edge_index_r0[0], pad_row]).reshape(ER, 128)
    col1 = jnp.concatenate([edge_index_r1[1], pad_col]).reshape(ER, 128)
    row1 = jnp.concatenate([edge_index_r1[0], pad_row]).reshape(ER, 128)

    deg0, deg1 = _deg_kernel(col0, col1)
    deg0 = deg0.reshape(DEG_P, 1)
    deg1 = deg1.reshape(DEG_P, 1)

    y10, y11 = _mm_l1(deg0, deg1, emb, W1_r0, W1_r1)
    s10, s11 = _scatter_kernel(y10, y11, col0, row0, col1, row1)
    y20, y21 = _mm_l2(deg0, deg1, s10, y10, s11, y11,
                      b1_r0.reshape(1, D), b1_r1.reshape(1, D), W2_r0, W2_r1)
    s20, s21 = _scatter_kernel(y20, y21, col0, row0, col1, row1)
    return _final(deg0, deg1, s20, y20, s21, y21,
                  b2_r0.reshape(1, D), b2_r1.reshape(1, D))


# speedup vs baseline: 1.3113x; 1.3113x over previous
import functools

import jax
import jax.numpy as jnp
from jax import lax
from jax.experimental import pallas as pl
from jax.experimental.pallas import tpu as pltpu
from jax.experimental.pallas import tpu_sc as plsc

N = 100000
E = 300000
D = 128

NC, NS = 2, 16
EP = 311296
ER = EP // 128
RPT = ER // NS
PAD_COL = 100223
C = 12544
NPASS = 4
ACC_ROWS = 12800
S_P = 2 * NPASS * C
DEG_P = 100352
DPT = DEG_P // NS
CH = 8
NCHUNK = RPT // CH
LIST_CAP = 1280
GB = 64
P_C0 = 5

_mesh = plsc.VectorSubcoreMesh(
    core_axis_name="c", subcore_axis_name="s", num_cores=NC, num_subcores=NS)


@functools.partial(
    pl.kernel,
    out_type=[jax.ShapeDtypeStruct((DEG_P,), jnp.float32),
              jax.ShapeDtypeStruct((DEG_P,), jnp.float32)],
    mesh=_mesh,
    compiler_params=pltpu.CompilerParams(needs_layout_passes=False),
    scratch_types=[
        pltpu.VMEM((RPT, 128), jnp.int32),
        pltpu.VMEM((128,), jnp.float32),
        pltpu.VMEM((DPT,), jnp.float32),
        pltpu.VMEM_SHARED((DEG_P,), jnp.float32),
    ],
)
def _deg_kernel(col0_hbm, col1_hbm, deg0_hbm, deg1_hbm,
                colbuf, ones_v, zero_v, acc):
    c = lax.axis_index("c")
    s = lax.axis_index("s")

    def fill_ones(i, _):
        ones_v[pl.ds(i * 16, 16)] = jnp.full((16,), 1.0, jnp.float32)
        return 0
    lax.fori_loop(0, 8, fill_ones, 0)

    def fill_zero(i, _):
        zero_v[pl.ds(i * 16, 16)] = jnp.zeros((16,), jnp.float32)
        return 0
    lax.fori_loop(0, DPT // 16, fill_zero, 0)
    pltpu.sync_copy(zero_v, acc.at[pl.ds(s * DPT, DPT)])

    @pl.when(c == 0)
    def _():
        pltpu.sync_copy(col0_hbm.at[pl.ds(s * RPT, RPT)], colbuf)

    @pl.when(c == 1)
    def _():
        pltpu.sync_copy(col1_hbm.at[pl.ds(s * RPT, RPT)], colbuf)

    plsc.subcore_barrier()

    def body(j, _):
        pltpu.sync_copy(ones_v, acc.at[colbuf.at[j]], add=True)
        return 0
    lax.fori_loop(0, RPT, body, 0)

    plsc.subcore_barrier()

    @pl.when(c == 0)
    def _():
        pltpu.sync_copy(acc.at[pl.ds(s * DPT, DPT)],
                        deg0_hbm.at[pl.ds(s * DPT, DPT)])

    @pl.when(c == 1)
    def _():
        pltpu.sync_copy(acc.at[pl.ds(s * DPT, DPT)],
                        deg1_hbm.at[pl.ds(s * DPT, DPT)])


@functools.partial(
    pl.kernel,
    out_type=[jax.ShapeDtypeStruct((S_P, D), jnp.float32),
              jax.ShapeDtypeStruct((S_P, D), jnp.float32)],
    mesh=_mesh,
    compiler_params=pltpu.CompilerParams(needs_layout_passes=False),
    scratch_types=[
        pltpu.VMEM((2, CH, 128), jnp.int32),
        pltpu.VMEM((2, CH, 128), jnp.int32),
        pltpu.VMEM((LIST_CAP,), jnp.int32),
        pltpu.VMEM((LIST_CAP,), jnp.int32),
        pltpu.VMEM((2, GB), jnp.int32),
        pltpu.VMEM((2, GB, D), jnp.float32),
        pltpu.VMEM_SHARED((ACC_ROWS, D), jnp.float32),
        pltpu.SemaphoreType.DMA,
        pltpu.SemaphoreType.DMA,
        pltpu.SemaphoreType.DMA,
        pltpu.SemaphoreType.DMA,
        pltpu.SemaphoreType.DMA,
        pltpu.SemaphoreType.DMA,
        pltpu.SemaphoreType.DMA,
    ],
)
def _scatter_kernel(y0_hbm, y1_hbm, col0_hbm, row0_hbm, col1_hbm, row1_hbm,
                    s0_hbm, s1_hbm,
                    colbuf, rowbuf, collist, rowlist, colstage, gbuf, acc,
                    cb_sem, rb_sem, g_sem0, g_sem1, s_sem0, s_sem1, z_sem):
    c = lax.axis_index("c")
    s = lax.axis_index("s")
    iota = lax.iota(jnp.int32, 16)
    mall = jnp.ones((16,), jnp.bool_)
    spt = s * (ACC_ROWS // NS)
    g_sems = (g_sem0, g_sem1)
    s_sems = (s_sem0, s_sem1)

    for rel in range(2):
        col_hbm = (col0_hbm, col1_hbm)[rel]
        row_hbm = (row0_hbm, row1_hbm)[rel]
        y_hbm = (y0_hbm, y1_hbm)[rel]
        out_hbm = (s0_hbm, s1_hbm)[rel]

        trip = jnp.where(c == 0, P_C0, 2 * NPASS - P_C0)
        start = jnp.where(c == 0, 0, P_C0)

        def one_pass(p, _):
            base = (start + p) * C

            with jax.named_scope("sc_zero"):
                def zg(i, _):
                    for sl in range(2):
                        for k in range(8):
                            gbuf[sl, i, pl.ds(k * 16, 16)] = \
                                jnp.zeros((16,), jnp.float32)
                    return 0
                lax.fori_loop(0, GB, zg, 0)
                zdescs = [pltpu.make_async_copy(
                    gbuf.at[0], acc.at[pl.ds(spt + k * GB, GB)], z_sem)
                    for k in range(12)]
                zdescs.append(pltpu.make_async_copy(
                    gbuf.at[0].at[pl.ds(0, 32)],
                    acc.at[pl.ds(spt + 12 * GB, 32)], z_sem))
                for d in zdescs:
                    d.start()
                for d in zdescs:
                    d.wait()
            for sl in range(2):
                for k in range(4):
                    colstage[sl, pl.ds(k * 16, 16)] = \
                        jnp.full((16,), C, jnp.int32)
                pltpu.make_async_copy(
                    gbuf.at[sl], acc.at[colstage.at[sl]],
                    s_sems[sl]).start(add=True)
            plsc.subcore_barrier()

            def stage_and_gather(off, sl):
                pltpu.make_async_copy(
                    gbuf.at[sl], acc.at[colstage.at[sl]],
                    s_sems[sl]).wait()
                for k in range(4):
                    colstage[sl, pl.ds(k * 16, 16)] = \
                        plsc.load_gather(collist, [off + k * 16 + iota])
                pltpu.async_copy(
                    y_hbm.at[rowlist.at[pl.ds(off, GB)]],
                    gbuf.at[sl], g_sems[sl])

            def scatter(sl):
                pltpu.make_async_copy(
                    y_hbm.at[rowlist.at[pl.ds(0, GB)]],
                    gbuf.at[sl], g_sems[sl]).wait()
                pltpu.make_async_copy(
                    gbuf.at[sl], acc.at[colstage.at[sl]],
                    s_sems[sl]).start(add=True)

            def drain_pair(b, _):
                off = b * (2 * GB)
                stage_and_gather(off, 0)
                stage_and_gather(off + GB, 1)
                scatter(0)
                scatter(1)
                return 0

            def chunk_body(ch, n):
                par = lax.rem(ch, 2)
                nxt = s * RPT + (ch + 1) * CH

                @pl.when(ch + 1 < NCHUNK)
                def _():
                    pltpu.async_copy(col_hbm.at[pl.ds(nxt, CH)],
                                     colbuf.at[1 - par], cb_sem)
                    pltpu.async_copy(row_hbm.at[pl.ds(nxt, CH)],
                                     rowbuf.at[1 - par], rb_sem)

                def crow(j, n):
                    for i in range(8):
                        cv = colbuf[par, j, pl.ds(i * 16, 16)]
                        rv = rowbuf[par, j, pl.ds(i * 16, 16)]
                        m = (cv >= base) & (cv < base + C)
                        mi = m.astype(jnp.int32)
                        pos = plsc.cumsum(mi)
                        idx = n + pos - 1
                        plsc.store_scatter(collist, [idx], cv - base, mask=m)
                        plsc.store_scatter(rowlist, [idx], rv, mask=m)
                        n = n + pos[15]
                    return n
                with jax.named_scope("sc_compact"):
                    n = lax.fori_loop(0, CH, crow, n)
                nb = n // (2 * GB)
                with jax.named_scope("sc_drain"):
                    lax.fori_loop(0, nb, drain_pair, 0)
                roff = nb * (2 * GB)
                for k in range(8):
                    v = plsc.load_gather(collist, [roff + k * 16 + iota])
                    w = plsc.load_gather(rowlist, [roff + k * 16 + iota])
                    plsc.store_scatter(collist, [k * 16 + iota], v, mask=mall)
                    plsc.store_scatter(rowlist, [k * 16 + iota], w, mask=mall)

                @pl.when(ch + 1 < NCHUNK)
                def _():
                    pltpu.make_async_copy(col_hbm.at[pl.ds(nxt, CH)],
                                          colbuf.at[1 - par], cb_sem).wait()
                    pltpu.make_async_copy(row_hbm.at[pl.ds(nxt, CH)],
                                          rowbuf.at[1 - par], rb_sem).wait()
                return n - roff

            pltpu.sync_copy(col_hbm.at[pl.ds(s * RPT, CH)], colbuf.at[0])
            pltpu.sync_copy(row_hbm.at[pl.ds(s * RPT, CH)], rowbuf.at[0])
            n = lax.fori_loop(0, NCHUNK, chunk_body, 0)

            for k in range(8):
                plsc.store_scatter(collist, [n + k * 16 + iota],
                                   jnp.full((16,), C, jnp.int32), mask=mall)
                plsc.store_scatter(rowlist, [n + k * 16 + iota],
                                   jnp.zeros((16,), jnp.int32), mask=mall)
            drain_pair(0, 0)
            for sl in range(2):
                pltpu.make_async_copy(
                    gbuf.at[sl], acc.at[colstage.at[sl]],
                    s_sems[sl]).wait()
            plsc.subcore_barrier()
            with jax.named_scope("sc_writeout"):
                woff = s * (C // NS)
                pltpu.sync_copy(acc.at[pl.ds(woff, C // NS)],
                                out_hbm.at[pl.ds(base + woff, C // NS)])
            plsc.subcore_barrier()
            return 0
        lax.fori_loop(0, trip, one_pass, 0)


BR = 1024
GRID = (DEG_P // BR,)


def _mm_l1(deg0, deg1, emb, w0, w1):
    def body(deg0_ref, deg1_ref, emb_ref, w0_ref, w1_ref, y0_ref, y1_ref):
        dinv0 = lax.rsqrt(deg0_ref[...] + 1.0)
        dinv1 = lax.rsqrt(deg1_ref[...] + 1.0)
        x = emb_ref[...]
        y0_ref[...] = jnp.dot(x, w0_ref[...],
                              preferred_element_type=jnp.float32) * dinv0
        y1_ref[...] = jnp.dot(x, w1_ref[...],
                              preferred_element_type=jnp.float32) * dinv1
    return pl.pallas_call(
        body,
        grid=GRID,
        in_specs=[
            pl.BlockSpec((BR, 1), lambda i: (i, 0)),
            pl.BlockSpec((BR, 1), lambda i: (i, 0)),
            pl.BlockSpec((BR, D), lambda i: (i, 0)),
            pl.BlockSpec((D, D), lambda i: (0, 0)),
            pl.BlockSpec((D, D), lambda i: (0, 0)),
        ],
        out_specs=[pl.BlockSpec((BR, D), lambda i: (i, 0))] * 2,
        out_shape=[jax.ShapeDtypeStruct((N, D), jnp.float32)] * 2,
    )(deg0, deg1, emb, w0, w1)


def _mm_l2(deg0, deg1, s0, y0, s1, y1, b0, b1, w0, w1):
    def body(deg0_ref, deg1_ref, s0_ref, y0_ref, s1_ref, y1_ref,
             b0_ref, b1_ref, w0_ref, w1_ref, o0_ref, o1_ref):
        dinv0 = lax.rsqrt(deg0_ref[...] + 1.0)
        dinv1 = lax.rsqrt(deg1_ref[...] + 1.0)
        h = 0.5 * (dinv0 * (s0_ref[...] + y0_ref[...]) + b0_ref[...]
                   + dinv1 * (s1_ref[...] + y1_ref[...]) + b1_ref[...])
        h = jnp.maximum(h, 0.0)
        o0_ref[...] = jnp.dot(h, w0_ref[...],
                              preferred_element_type=jnp.float32) * dinv0
        o1_ref[...] = jnp.dot(h, w1_ref[...],
                              preferred_element_type=jnp.float32) * dinv1
    return pl.pallas_call(
        body,
        grid=GRID,
        in_specs=[
            pl.BlockSpec((BR, 1), lambda i: (i, 0)),
            pl.BlockSpec((BR, 1), lambda i: (i, 0)),
            pl.BlockSpec((BR, D), lambda i: (i, 0)),
            pl.BlockSpec((BR, D), lambda i: (i, 0)),
            pl.BlockSpec((BR, D), lambda i: (i, 0)),
            pl.BlockSpec((BR, D), lambda i: (i, 0)),
            pl.BlockSpec((1, D), lambda i: (0, 0)),
            pl.BlockSpec((1, D), lambda i: (0, 0)),
            pl.BlockSpec((D, D), lambda i: (0, 0)),
            pl.BlockSpec((D, D), lambda i: (0, 0)),
        ],
        out_specs=[pl.BlockSpec((BR, D), lambda i: (i, 0))] * 2,
        out_shape=[jax.ShapeDtypeStruct((N, D), jnp.float32)] * 2,
    )(deg0, deg1, s0, y0, s1, y1, b0, b1, w0, w1)


def _final(deg0, deg1, s0, y0, s1, y1, b0, b1):
    def body(deg0_ref, deg1_ref, s0_ref, y0_ref, s1_ref, y1_ref,
             b0_ref, b1_ref, o_ref):
        dinv0 = lax.rsqrt(deg0_ref[...] + 1.0)
        dinv1 = lax.rsqrt(deg1_ref[...] + 1.0)
        o_ref[...] = 0.5 * (dinv0 * (s0_ref[...] + y0_ref[...]) + b0_ref[...]
                            + dinv1 * (s1_ref[...] + y1_ref[...])
                            + b1_ref[...])
    return pl.pallas_call(
        body,
        grid=GRID,
        in_specs=[
            pl.BlockSpec((BR, 1), lambda i: (i, 0)),
            pl.BlockSpec((BR, 1), lambda i: (i, 0)),
            pl.BlockSpec((BR, D), lambda i: (i, 0)),
            pl.BlockSpec((BR, D), lambda i: (i, 0)),
            pl.BlockSpec((BR, D), lambda i: (i, 0)),
            pl.BlockSpec((BR, D), lambda i: (i, 0)),
            pl.BlockSpec((1, D), lambda i: (0, 0)),
            pl.BlockSpec((1, D), lambda i: (0, 0)),
        ],
        out_specs=pl.BlockSpec((BR, D), lambda i: (i, 0)),
        out_shape=jax.ShapeDtypeStruct((N, D), jnp.float32),
    )(deg0, deg1, s0, y0, s1, y1, b0, b1)


def kernel(edge_index_r0, edge_index_r1, emb, W1_r0, b1_r0, W1_r1, b1_r1,
           W2_r0, b2_r0, W2_r1, b2_r1):
    pad_col = jnp.full((EP - E,), PAD_COL, jnp.int32)
    pad_row = jnp.zeros((EP - E,), jnp.int32)
    col0 = jnp.concatenate([edge_index_r0[1], pad_col]).reshape(ER, 128)
    row0 = jnp.concatenate([edge_index_r0[0], pad_row]).reshape(ER, 128)
    col1 = jnp.concatenate([edge_index_r1[1], pad_col]).reshape(ER, 128)
    row1 = jnp.concatenate([edge_index_r1[0], pad_row]).reshape(ER, 128)

    deg0, deg1 = _deg_kernel(col0, col1)
    deg0 = deg0.reshape(DEG_P, 1)
    deg1 = deg1.reshape(DEG_P, 1)

    y10, y11 = _mm_l1(deg0, deg1, emb, W1_r0, W1_r1)
    s10, s11 = _scatter_kernel(y10, y11, col0, row0, col1, row1)
    y20, y21 = _mm_l2(deg0, deg1, s10, y10, s11, y11,
                      b1_r0.reshape(1, D), b1_r1.reshape(1, D), W2_r0, W2_r1)
    s20, s21 = _scatter_kernel(y20, y21, col0, row0, col1, row1)
    return _final(deg0, deg1, s20, y20, s21, y21,
                  b2_r0.reshape(1, D), b2_r1.reshape(1, D))


# speedup vs baseline: 1.3825x; 1.0542x over previous
import functools

import jax
import jax.numpy as jnp
from jax import lax
from jax.experimental import pallas as pl
from jax.experimental.pallas import tpu as pltpu
from jax.experimental.pallas import tpu_sc as plsc

N = 100000
E = 300000
D = 128

NC, NS = 2, 16
EP = 311296
ER = EP // 128
RPT = ER // NS
PAD_COL = 100223
C = 12544
NPASS = 4
ACC_ROWS = 12800
S_P = 2 * NPASS * C
DEG_P = 100352
DPT = DEG_P // NS
CH = 8
NCHUNK = RPT // CH
LIST_CAP = 1280
GB = 64
P_C0 = 7

_mesh = plsc.VectorSubcoreMesh(
    core_axis_name="c", subcore_axis_name="s", num_cores=NC, num_subcores=NS)


@functools.partial(
    pl.kernel,
    out_type=[jax.ShapeDtypeStruct((DEG_P,), jnp.float32),
              jax.ShapeDtypeStruct((DEG_P,), jnp.float32)],
    mesh=_mesh,
    compiler_params=pltpu.CompilerParams(needs_layout_passes=False),
    scratch_types=[
        pltpu.VMEM((RPT, 128), jnp.int32),
        pltpu.VMEM((128,), jnp.float32),
        pltpu.VMEM((DPT,), jnp.float32),
        pltpu.VMEM_SHARED((DEG_P,), jnp.float32),
    ],
)
def _deg_kernel(col0_hbm, col1_hbm, deg0_hbm, deg1_hbm,
                colbuf, ones_v, zero_v, acc):
    c = lax.axis_index("c")
    s = lax.axis_index("s")

    def fill_ones(i, _):
        ones_v[pl.ds(i * 16, 16)] = jnp.full((16,), 1.0, jnp.float32)
        return 0
    lax.fori_loop(0, 8, fill_ones, 0)

    def fill_zero(i, _):
        zero_v[pl.ds(i * 16, 16)] = jnp.zeros((16,), jnp.float32)
        return 0
    lax.fori_loop(0, DPT // 16, fill_zero, 0)
    pltpu.sync_copy(zero_v, acc.at[pl.ds(s * DPT, DPT)])

    @pl.when(c == 0)
    def _():
        pltpu.sync_copy(col0_hbm.at[pl.ds(s * RPT, RPT)], colbuf)

    @pl.when(c == 1)
    def _():
        pltpu.sync_copy(col1_hbm.at[pl.ds(s * RPT, RPT)], colbuf)

    plsc.subcore_barrier()

    def body(j, _):
        pltpu.sync_copy(ones_v, acc.at[colbuf.at[j]], add=True)
        return 0
    lax.fori_loop(0, RPT, body, 0)

    plsc.subcore_barrier()

    @pl.when(c == 0)
    def _():
        pltpu.sync_copy(acc.at[pl.ds(s * DPT, DPT)],
                        deg0_hbm.at[pl.ds(s * DPT, DPT)])

    @pl.when(c == 1)
    def _():
        pltpu.sync_copy(acc.at[pl.ds(s * DPT, DPT)],
                        deg1_hbm.at[pl.ds(s * DPT, DPT)])


@functools.partial(
    pl.kernel,
    out_type=[jax.ShapeDtypeStruct((S_P, D), jnp.float32),
              jax.ShapeDtypeStruct((S_P, D), jnp.float32)],
    mesh=_mesh,
    compiler_params=pltpu.CompilerParams(needs_layout_passes=False),
    scratch_types=[
        pltpu.VMEM((2, CH, 128), jnp.int32),
        pltpu.VMEM((2, CH, 128), jnp.int32),
        pltpu.VMEM((LIST_CAP,), jnp.int32),
        pltpu.VMEM((LIST_CAP,), jnp.int32),
        pltpu.VMEM((2, GB), jnp.int32),
        pltpu.VMEM((2, GB, D), jnp.float32),
        pltpu.VMEM_SHARED((ACC_ROWS, D), jnp.float32),
        pltpu.SemaphoreType.DMA,
        pltpu.SemaphoreType.DMA,
        pltpu.SemaphoreType.DMA,
        pltpu.SemaphoreType.DMA,
        pltpu.SemaphoreType.DMA,
        pltpu.SemaphoreType.DMA,
        pltpu.SemaphoreType.DMA,
    ],
)
def _scatter_kernel(y0_hbm, y1_hbm, col0_hbm, row0_hbm, col1_hbm, row1_hbm,
                    s0_hbm, s1_hbm,
                    colbuf, rowbuf, collist, rowlist, colstage, gbuf, acc,
                    cb_sem, rb_sem, g_sem0, g_sem1, s_sem0, s_sem1, z_sem):
    c = lax.axis_index("c")
    s = lax.axis_index("s")
    iota = lax.iota(jnp.int32, 16)
    mall = jnp.ones((16,), jnp.bool_)
    spt = s * (ACC_ROWS // NS)
    g_sems = (g_sem0, g_sem1)
    s_sems = (s_sem0, s_sem1)

    for rel in range(2):
        col_hbm = (col0_hbm, col1_hbm)[rel]
        row_hbm = (row0_hbm, row1_hbm)[rel]
        y_hbm = (y0_hbm, y1_hbm)[rel]
        out_hbm = (s0_hbm, s1_hbm)[rel]

        trip = jnp.where(c == 0, P_C0, 2 * NPASS - P_C0)
        start = jnp.where(c == 0, 0, P_C0)

        def one_pass(p, _):
            base = (start + p) * C

            with jax.named_scope("sc_zero"):
                def zg(i, _):
                    for sl in range(2):
                        for k in range(8):
                            gbuf[sl, i, pl.ds(k * 16, 16)] = \
                                jnp.zeros((16,), jnp.float32)
                    return 0
                lax.fori_loop(0, GB, zg, 0)
                zdescs = [pltpu.make_async_copy(
                    gbuf.at[0], acc.at[pl.ds(spt + k * GB, GB)], z_sem)
                    for k in range(12)]
                zdescs.append(pltpu.make_async_copy(
                    gbuf.at[0].at[pl.ds(0, 32)],
                    acc.at[pl.ds(spt + 12 * GB, 32)], z_sem))
                for d in zdescs:
                    d.start()
                for d in zdescs:
                    d.wait()
            for sl in range(2):
                for k in range(4):
                    colstage[sl, pl.ds(k * 16, 16)] = \
                        jnp.full((16,), C, jnp.int32)
                pltpu.make_async_copy(
                    gbuf.at[sl], acc.at[colstage.at[sl]],
                    s_sems[sl]).start(add=True)
            plsc.subcore_barrier()

            def stage_and_gather(off, sl):
                pltpu.make_async_copy(
                    gbuf.at[sl], acc.at[colstage.at[sl]],
                    s_sems[sl]).wait()
                for k in range(4):
                    colstage[sl, pl.ds(k * 16, 16)] = \
                        plsc.load_gather(collist, [off + k * 16 + iota])
                pltpu.async_copy(
                    y_hbm.at[rowlist.at[pl.ds(off, GB)]],
                    gbuf.at[sl], g_sems[sl])

            def scatter(sl):
                pltpu.make_async_copy(
                    y_hbm.at[rowlist.at[pl.ds(0, GB)]],
                    gbuf.at[sl], g_sems[sl]).wait()
                pltpu.make_async_copy(
                    gbuf.at[sl], acc.at[colstage.at[sl]],
                    s_sems[sl]).start(add=True)

            def drain_pair(b, _):
                off = b * (2 * GB)
                stage_and_gather(off, 0)
                stage_and_gather(off + GB, 1)
                scatter(0)
                scatter(1)
                return 0

            def chunk_body(ch, n):
                par = lax.rem(ch, 2)
                nxt = s * RPT + (ch + 1) * CH

                @pl.when(ch + 1 < NCHUNK)
                def _():
                    pltpu.async_copy(col_hbm.at[pl.ds(nxt, CH)],
                                     colbuf.at[1 - par], cb_sem)
                    pltpu.async_copy(row_hbm.at[pl.ds(nxt, CH)],
                                     rowbuf.at[1 - par], rb_sem)

                def crow(j, n):
                    for i in range(8):
                        cv = colbuf[par, j, pl.ds(i * 16, 16)]
                        rv = rowbuf[par, j, pl.ds(i * 16, 16)]
                        m = (cv >= base) & (cv < base + C)
                        mi = m.astype(jnp.int32)
                        pos = plsc.cumsum(mi)
                        idx = n + pos - 1
                        plsc.store_scatter(collist, [idx], cv - base, mask=m)
                        plsc.store_scatter(rowlist, [idx], rv, mask=m)
                        n = n + pos[15]
                    return n
                with jax.named_scope("sc_compact"):
                    n = lax.fori_loop(0, CH, crow, n)
                nb = n // (2 * GB)
                with jax.named_scope("sc_drain"):
                    lax.fori_loop(0, nb, drain_pair, 0)
                roff = nb * (2 * GB)
                for k in range(8):
                    v = plsc.load_gather(collist, [roff + k * 16 + iota])
                    w = plsc.load_gather(rowlist, [roff + k * 16 + iota])
                    plsc.store_scatter(collist, [k * 16 + iota], v, mask=mall)
                    plsc.store_scatter(rowlist, [k * 16 + iota], w, mask=mall)

                @pl.when(ch + 1 < NCHUNK)
                def _():
                    pltpu.make_async_copy(col_hbm.at[pl.ds(nxt, CH)],
                                          colbuf.at[1 - par], cb_sem).wait()
                    pltpu.make_async_copy(row_hbm.at[pl.ds(nxt, CH)],
                                          rowbuf.at[1 - par], rb_sem).wait()
                return n - roff

            pltpu.sync_copy(col_hbm.at[pl.ds(s * RPT, CH)], colbuf.at[0])
            pltpu.sync_copy(row_hbm.at[pl.ds(s * RPT, CH)], rowbuf.at[0])
            n = lax.fori_loop(0, NCHUNK, chunk_body, 0)

            for k in range(8):
                plsc.store_scatter(collist, [n + k * 16 + iota],
                                   jnp.full((16,), C, jnp.int32), mask=mall)
                plsc.store_scatter(rowlist, [n + k * 16 + iota],
                                   jnp.zeros((16,), jnp.int32), mask=mall)
            drain_pair(0, 0)
            for sl in range(2):
                pltpu.make_async_copy(
                    gbuf.at[sl], acc.at[colstage.at[sl]],
                    s_sems[sl]).wait()
            plsc.subcore_barrier()
            with jax.named_scope("sc_writeout"):
                woff = s * (C // NS)
                pltpu.sync_copy(acc.at[pl.ds(woff, C // NS)],
                                out_hbm.at[pl.ds(base + woff, C // NS)])
            plsc.subcore_barrier()
            return 0
        lax.fori_loop(0, trip, one_pass, 0)


BR = 1024
GRID = (DEG_P // BR,)


def _mm_l1(deg0, deg1, emb, w0, w1):
    def body(deg0_ref, deg1_ref, emb_ref, w0_ref, w1_ref, y0_ref, y1_ref):
        dinv0 = lax.rsqrt(deg0_ref[...] + 1.0)
        dinv1 = lax.rsqrt(deg1_ref[...] + 1.0)
        x = emb_ref[...]
        y0_ref[...] = jnp.dot(x, w0_ref[...],
                              preferred_element_type=jnp.float32) * dinv0
        y1_ref[...] = jnp.dot(x, w1_ref[...],
                              preferred_element_type=jnp.float32) * dinv1
    return pl.pallas_call(
        body,
        grid=GRID,
        in_specs=[
            pl.BlockSpec((BR, 1), lambda i: (i, 0)),
            pl.BlockSpec((BR, 1), lambda i: (i, 0)),
            pl.BlockSpec((BR, D), lambda i: (i, 0)),
            pl.BlockSpec((D, D), lambda i: (0, 0)),
            pl.BlockSpec((D, D), lambda i: (0, 0)),
        ],
        out_specs=[pl.BlockSpec((BR, D), lambda i: (i, 0))] * 2,
        out_shape=[jax.ShapeDtypeStruct((N, D), jnp.float32)] * 2,
    )(deg0, deg1, emb, w0, w1)


def _mm_l2(deg0, deg1, s0, y0, s1, y1, b0, b1, w0, w1):
    def body(deg0_ref, deg1_ref, s0_ref, y0_ref, s1_ref, y1_ref,
             b0_ref, b1_ref, w0_ref, w1_ref, o0_ref, o1_ref):
        dinv0 = lax.rsqrt(deg0_ref[...] + 1.0)
        dinv1 = lax.rsqrt(deg1_ref[...] + 1.0)
        h = 0.5 * (dinv0 * (s0_ref[...] + y0_ref[...]) + b0_ref[...]
                   + dinv1 * (s1_ref[...] + y1_ref[...]) + b1_ref[...])
        h = jnp.maximum(h, 0.0)
        o0_ref[...] = jnp.dot(h, w0_ref[...],
                              preferred_element_type=jnp.float32) * dinv0
        o1_ref[...] = jnp.dot(h, w1_ref[...],
                              preferred_element_type=jnp.float32) * dinv1
    return pl.pallas_call(
        body,
        grid=GRID,
        in_specs=[
            pl.BlockSpec((BR, 1), lambda i: (i, 0)),
            pl.BlockSpec((BR, 1), lambda i: (i, 0)),
            pl.BlockSpec((BR, D), lambda i: (i, 0)),
            pl.BlockSpec((BR, D), lambda i: (i, 0)),
            pl.BlockSpec((BR, D), lambda i: (i, 0)),
            pl.BlockSpec((BR, D), lambda i: (i, 0)),
            pl.BlockSpec((1, D), lambda i: (0, 0)),
            pl.BlockSpec((1, D), lambda i: (0, 0)),
            pl.BlockSpec((D, D), lambda i: (0, 0)),
            pl.BlockSpec((D, D), lambda i: (0, 0)),
        ],
        out_specs=[pl.BlockSpec((BR, D), lambda i: (i, 0))] * 2,
        out_shape=[jax.ShapeDtypeStruct((N, D), jnp.float32)] * 2,
    )(deg0, deg1, s0, y0, s1, y1, b0, b1, w0, w1)


def _final(deg0, deg1, s0, y0, s1, y1, b0, b1):
    def body(deg0_ref, deg1_ref, s0_ref, y0_ref, s1_ref, y1_ref,
             b0_ref, b1_ref, o_ref):
        dinv0 = lax.rsqrt(deg0_ref[...] + 1.0)
        dinv1 = lax.rsqrt(deg1_ref[...] + 1.0)
        o_ref[...] = 0.5 * (dinv0 * (s0_ref[...] + y0_ref[...]) + b0_ref[...]
                            + dinv1 * (s1_ref[...] + y1_ref[...])
                            + b1_ref[...])
    return pl.pallas_call(
        body,
        grid=GRID,
        in_specs=[
            pl.BlockSpec((BR, 1), lambda i: (i, 0)),
            pl.BlockSpec((BR, 1), lambda i: (i, 0)),
            pl.BlockSpec((BR, D), lambda i: (i, 0)),
            pl.BlockSpec((BR, D), lambda i: (i, 0)),
            pl.BlockSpec((BR, D), lambda i: (i, 0)),
            pl.BlockSpec((BR, D), lambda i: (i, 0)),
            pl.BlockSpec((1, D), lambda i: (0, 0)),
            pl.BlockSpec((1, D), lambda i: (0, 0)),
        ],
        out_specs=pl.BlockSpec((BR, D), lambda i: (i, 0)),
        out_shape=jax.ShapeDtypeStruct((N, D), jnp.float32),
    )(deg0, deg1, s0, y0, s1, y1, b0, b1)


def kernel(edge_index_r0, edge_index_r1, emb, W1_r0, b1_r0, W1_r1, b1_r1,
           W2_r0, b2_r0, W2_r1, b2_r1):
    pad_col = jnp.full((EP - E,), PAD_COL, jnp.int32)
    pad_row = jnp.zeros((EP - E,), jnp.int32)
    col0 = jnp.concatenate([edge_index_r0[1], pad_col]).reshape(ER, 128)
    row0 = jnp.concatenate([edge_index_r0[0], pad_row]).reshape(ER, 128)
    col1 = jnp.concatenate([edge_index_r1[1], pad_col]).reshape(ER, 128)
    row1 = jnp.concatenate([edge_index_r1[0], pad_row]).reshape(ER, 128)

    deg0, deg1 = _deg_kernel(col0, col1)
    deg0 = deg0.reshape(DEG_P, 1)
    deg1 = deg1.reshape(DEG_P, 1)

    y10, y11 = _mm_l1(deg0, deg1, emb, W1_r0, W1_r1)
    s10, s11 = _scatter_kernel(y10, y11, col0, row0, col1, row1)
    y20, y21 = _mm_l2(deg0, deg1, s10, y10, s11, y11,
                      b1_r0.reshape(1, D), b1_r1.reshape(1, D), W2_r0, W2_r1)
    s20, s21 = _scatter_kernel(y20, y21, col0, row0, col1, row1)
    return _final(deg0, deg1, s20, y20, s21, y21,
                  b2_r0.reshape(1, D), b2_r1.reshape(1, D))
